# Initial kernel scaffold; baseline (speedup 1.0000x reference)
import jax
import jax.numpy as jnp
from jax.experimental import pallas as pl


def kernel(node_features, edge_features, adj_e, adj_v, T, edge_index, edge_type, W_rel, W_self, b_self):
    raise NotImplementedError("write your pallas kernel here")


# speedup vs baseline: 21.2913x; 21.2913x over previous
import functools

import jax
import jax.numpy as jnp
from jax import lax
from jax.experimental import pallas as pl
from jax.experimental.pallas import tpu as pltpu
from jax.experimental.pallas import tpu_sc as plsc

NC = 2
NS = 16
NW = NC * NS

CH = 80


def _transform_body(x_ref, w_ref, b_ref, o_ref, *, num_rel):
  q = pl.program_id(0)
  acc = lax.dot_general(
      x_ref[...], w_ref[0],
      dimension_numbers=(((1,), (1,)), ((), ())),
      preferred_element_type=jnp.float32,
  )

  @pl.when(q == num_rel)
  def _():
    o_ref[0] = acc + b_ref[...]

  @pl.when(q != num_rel)
  def _():
    o_ref[0] = acc


def _combine_body(p_ref, s_ref, o_ref):
  o_ref[...] = p_ref[0] + p_ref[1] + s_ref[...]


def _make_sc_kernel(n, e, d, r):
  epw = e // NW
  nch = epw // CH
  ept = e // NS
  nab = ept // 2000
  deg_sz = r * n
  nrc = n // CH

  mesh = plsc.VectorSubcoreMesh(core_axis_name="c", subcore_axis_name="s")

  @functools.partial(
      pl.kernel,
      mesh=mesh,
      compiler_params=pltpu.CompilerParams(needs_layout_passes=False),
      out_type=jax.ShapeDtypeStruct((NC, n, d), jnp.float32),
      scratch_types=[
          pltpu.VMEM_SHARED((deg_sz,), jnp.float32),
          pltpu.VMEM_SHARED((n, d), jnp.float32),
          pltpu.VMEM((epw,), jnp.float32),
          pltpu.VMEM((8000,), jnp.float32),
          pltpu.VMEM((2000,), jnp.int32),
          pltpu.VMEM((2000,), jnp.int32),
          pltpu.VMEM((25, CH), jnp.int32),
          pltpu.VMEM((25, CH), jnp.float32),
          pltpu.VMEM((CH, d), jnp.float32),
          pltpu.VMEM((CH,), jnp.float32),
          pltpu.VMEM((CH,), jnp.int32),
          pltpu.SemaphoreType.DMA,
          pltpu.SemaphoreType.DMA,
      ],
  )
  def sc_kernel(trel, et_h, tgt_h, row_h, ef_h, out_hbm,
                deg_sp, out_sp, c_all,
                ef_blk, et_a, tgt_a, key_a, wabs_a, rows, deg80, srow80,
                sem_g, sem_s):
    cid = lax.axis_index("c")
    sid = lax.axis_index("s")
    wid = sid * NC + cid
    lane4 = lax.iota(jnp.int32, 16) * 4

    def zrow(i, carry):
      for h in range(d // 16):
        rows[i, pl.ds(h * 16, 16)] = jnp.zeros((16,), jnp.float32)
      return carry

    lax.fori_loop(0, CH, zrow, 0)

    def zout(k, carry):
      ch_id = sid + k * NS

      @pl.when(ch_id < nrc)
      def _():
        pltpu.sync_copy(rows, out_sp.at[pl.ds(ch_id * CH, CH)])

      return carry

    lax.fori_loop(0, -(-nrc // NS), zout, 0)

    def zc(i, carry):
      c_all[pl.ds(i * 16, 16)] = jnp.zeros((16,), jnp.float32)
      return carry

    lax.fori_loop(0, 250, zc, 0)

    @pl.when(sid < 10)
    def _():
      pltpu.sync_copy(c_all.at[pl.ds(0, 4000)],
                      deg_sp.at[pl.ds(sid * 4000, 4000)])

    plsc.subcore_barrier()

    def ablock(b, carry):
      abase = sid * ept + b * 2000
      pltpu.sync_copy(et_h.at[pl.ds(abase, 2000)], et_a)
      pltpu.sync_copy(tgt_h.at[pl.ds(abase, 2000)], tgt_a)
      pltpu.sync_copy(ef_h.at[pl.ds(abase * 4, 8000)], ef_blk)

      def rowloop(rr, c2):
        for g in range(5):
          f = rr * CH + g * 16
          et16 = et_a[pl.ds(f, 16)]
          tg16 = tgt_a[pl.ds(f, 16)]
          key_a[rr, pl.ds(g * 16, 16)] = et16 * n + tg16
          acc = plsc.load_gather(ef_blk, [lane4 + f * 4])
          for j in range(1, 4):
            acc = acc + plsc.load_gather(ef_blk, [lane4 + (f * 4 + j)])
          wabs_a[rr, pl.ds(g * 16, 16)] = jnp.abs(acc * 0.25)
        return c2

      lax.fori_loop(0, 25, rowloop, 0)
      descs = [
          pltpu.async_copy(wabs_a.at[j], deg_sp.at[key_a.at[j]], sem_s,
                           add=True)
          for j in range(25)
      ]
      for desc in descs:
        desc.wait()
      return carry

    lax.fori_loop(0, nab, ablock, 0)
    plsc.subcore_barrier()

    base = wid * epw

    def bblock(b, carry):
      bb = base + b * 2000
      pltpu.sync_copy(et_h.at[pl.ds(bb, 2000)], et_a)
      pltpu.sync_copy(tgt_h.at[pl.ds(bb, 2000)], tgt_a)
      pltpu.sync_copy(ef_h.at[pl.ds(bb * 4, 8000)], ef_blk)

      def rowloop(rr, c2):
        for g in range(5):
          fb = rr * CH + g * 16
          et16 = et_a[pl.ds(fb, 16)]
          tg16 = tgt_a[pl.ds(fb, 16)]
          key_a[rr, pl.ds(g * 16, 16)] = et16 * n + tg16
          acc = plsc.load_gather(ef_blk, [lane4 + fb * 4])
          for j in range(1, 4):
            acc = acc + plsc.load_gather(ef_blk, [lane4 + (fb * 4 + j)])
          c_all[pl.ds(b * 2000 + fb, 16)] = acc * 0.25
        return c2

      lax.fori_loop(0, 25, rowloop, 0)

      def normloop(rr, c2):
        pltpu.sync_copy(deg_sp.at[key_a.at[rr]], deg80)
        for g in range(5):
          sl = pl.ds(b * 2000 + rr * CH + g * 16, 16)
          c_all[sl] = c_all[sl] / (deg80[pl.ds(g * 16, 16)] + 1e-8)
        return c2

      lax.fori_loop(0, 25, normloop, 0)
      return carry

    lax.fori_loop(0, 5, bblock, 0)

    def cblock(b, carry):
      bb = base + b * 2000
      pltpu.sync_copy(et_h.at[pl.ds(bb, 2000)], et_a)
      pltpu.sync_copy(tgt_h.at[pl.ds(bb, 2000)], tgt_a)

      def cchunk(rr, c2):
        for g in range(5):
          fb = rr * CH + g * 16
          et16 = et_a[pl.ds(fb, 16)]
          tg16 = tgt_a[pl.ds(fb, 16)]
          key_a[rr, pl.ds(g * 16, 16)] = et16 * n + tg16
        pltpu.sync_copy(row_h.at[pl.ds(bb + rr * CH, CH)], srow80)
        pltpu.async_copy(trel.at[key_a.at[rr]], rows, sem_g).wait()

        def scale(g, c3):
          c16 = c_all[pl.ds(b * 2000 + rr * CH + g * 16, 16)]
          for k in range(16):
            spl = jnp.take(c16, jnp.full((16,), k, jnp.int32), mode="fill")
            erow = g * 16 + k
            for h in range(d // 16):
              sl = pl.ds(h * 16, 16)
              rows[erow, sl] = rows[erow, sl] * spl
          return c3

        lax.fori_loop(0, CH // 16, scale, 0)
        pltpu.sync_copy(rows, out_sp.at[srow80], add=True)
        return c2

      lax.fori_loop(0, 25, cchunk, 0)
      return carry

    lax.fori_loop(0, 5, cblock, 0)
    plsc.subcore_barrier()

    def wb(k, carry):
      ch_id = sid + k * NS

      @pl.when(ch_id < nrc)
      def _():
        pltpu.sync_copy(out_sp.at[pl.ds(ch_id * CH, CH)], rows)
        pltpu.sync_copy(rows, out_hbm.at[cid, pl.ds(ch_id * CH, CH)])

      return carry

    lax.fori_loop(0, -(-nrc // NS), wb, 0)

  return sc_kernel


def kernel(node_features, edge_features, adj_e, adj_v, T, edge_index,
           edge_type, W_rel, W_self, b_self):
  n = adj_v.shape[0]
  e = edge_index.shape[1]
  din = node_features.shape[1]
  dout = W_self.shape[0]
  r = W_rel.shape[0]

  w_stack = jnp.concatenate([W_rel, W_self[None]], axis=0)
  bm = 2000
  t_all = pl.pallas_call(
      functools.partial(_transform_body, num_rel=r),
      grid=(r + 1, n // bm),
      in_specs=[
          pl.BlockSpec((bm, din), lambda q, i: (i, 0)),
          pl.BlockSpec((1, dout, din), lambda q, i: (q, 0, 0)),
          pl.BlockSpec((1, dout), lambda q, i: (0, 0)),
      ],
      out_specs=pl.BlockSpec((1, bm, dout), lambda q, i: (q, i, 0)),
      out_shape=jax.ShapeDtypeStruct((r + 1, n, dout), jnp.float32),
  )(node_features, w_stack, b_self.reshape(1, dout))

  trel = t_all[:r].reshape(r * n, dout)
  self_out = t_all[r]

  row_flat = edge_index[0]
  tgt_flat = edge_index[1]
  ef_flat = edge_features.reshape(-1)
  partials = _make_sc_kernel(n, e, dout, r)(
      trel, edge_type, tgt_flat, row_flat, ef_flat)

  out = pl.pallas_call(
      _combine_body,
      grid=(n // bm,),
      in_specs=[
          pl.BlockSpec((NC, bm, dout), lambda i: (0, i, 0)),
          pl.BlockSpec((bm, dout), lambda i: (i, 0)),
      ],
      out_specs=pl.BlockSpec((bm, dout), lambda i: (i, 0)),
      out_shape=jax.ShapeDtypeStruct((n, dout), jnp.float32),
  )(partials, self_out)

  return out, edge_features


# speedup vs baseline: 29.3523x; 1.3786x over previous
import functools

import jax
import jax.numpy as jnp
from jax import lax
from jax.experimental import pallas as pl
from jax.experimental.pallas import tpu as pltpu
from jax.experimental.pallas import tpu_sc as plsc

NC = 2
NS = 16
NW = NC * NS

CH = 80
BLK = 2000
CPB = BLK // CH


def _transform_body(x_ref, w_ref, b_ref, o_ref, *, num_rel):
  q = pl.program_id(0)
  acc = lax.dot_general(
      x_ref[...], w_ref[0],
      dimension_numbers=(((1,), (1,)), ((), ())),
      preferred_element_type=jnp.float32,
  )

  @pl.when(q == num_rel)
  def _():
    o_ref[0] = acc + b_ref[...]

  @pl.when(q != num_rel)
  def _():
    o_ref[0] = acc


def _combine_body(p_ref, s_ref, o_ref):
  o_ref[...] = p_ref[0] + p_ref[1] + s_ref[...]


def _make_sc_kernel(n, e, d, r):
  epw = e // NW
  nch = epw // CH
  nbl = epw // BLK
  ept = e // NS
  nab = ept // BLK
  deg_sz = r * n
  nrc = n // CH

  mesh = plsc.VectorSubcoreMesh(core_axis_name="c", subcore_axis_name="s")

  @functools.partial(
      pl.kernel,
      mesh=mesh,
      compiler_params=pltpu.CompilerParams(needs_layout_passes=False),
      out_type=jax.ShapeDtypeStruct((NC, n, d), jnp.float32),
      scratch_types=[
          pltpu.VMEM_SHARED((deg_sz,), jnp.float32),
          pltpu.VMEM_SHARED((n, d), jnp.float32),
          pltpu.VMEM((BLK * 4,), jnp.float32),
          pltpu.VMEM((BLK,), jnp.int32),
          pltpu.VMEM((BLK,), jnp.int32),
          pltpu.VMEM((CPB, CH), jnp.int32),
          pltpu.VMEM((CPB, CH), jnp.float32),
          pltpu.VMEM((CH, d), jnp.float32),
          pltpu.VMEM((CH, d), jnp.float32),
          pltpu.VMEM((CH,), jnp.float32),
          pltpu.VMEM((CH,), jnp.float32),
          pltpu.VMEM((CH,), jnp.int32),
          pltpu.VMEM((CH,), jnp.int32),
          pltpu.VMEM((CH,), jnp.float32),
          pltpu.SemaphoreType.DMA,
          pltpu.SemaphoreType.DMA,
          pltpu.SemaphoreType.DMA,
          pltpu.SemaphoreType.DMA,
          pltpu.SemaphoreType.DMA,
      ],
  )
  def sc_kernel(trel, et_h, tgt_h, row_h, ef_h, out_hbm,
                deg_sp, out_sp,
                ef_blk, et_a, tgt_a, key_a, wabs_a,
                rows0, rows1, c80_0, c80_1, srow0, srow1, deg80,
                sem_g0, sem_g1, sem_s0, sem_s1, sem_a):
    cid = lax.axis_index("c")
    sid = lax.axis_index("s")
    wid = sid * NC + cid
    lane4 = lax.iota(jnp.int32, 16) * 4

    def zrow(i, carry):
      for h in range(d // 16):
        rows0[i, pl.ds(h * 16, 16)] = jnp.zeros((16,), jnp.float32)
      return carry

    lax.fori_loop(0, CH, zrow, 0)

    def zout(k, carry):
      ch_id = sid + k * NS

      @pl.when(ch_id < nrc)
      def _():
        pltpu.sync_copy(rows0, out_sp.at[pl.ds(ch_id * CH, CH)])

      return carry

    lax.fori_loop(0, -(-nrc // NS), zout, 0)

    def zc(i, carry):
      ef_blk[pl.ds(i * 16, 16)] = jnp.zeros((16,), jnp.float32)
      return carry

    lax.fori_loop(0, 250, zc, 0)

    @pl.when(sid < 10)
    def _():
      pltpu.sync_copy(ef_blk.at[pl.ds(0, 4000)],
                      deg_sp.at[pl.ds(sid * 4000, 4000)])

    plsc.subcore_barrier()

    def ablock(b, carry):
      abase = sid * ept + b * BLK
      pltpu.sync_copy(et_h.at[pl.ds(abase, BLK)], et_a)
      pltpu.sync_copy(tgt_h.at[pl.ds(abase, BLK)], tgt_a)
      pltpu.sync_copy(ef_h.at[pl.ds(abase * 4, BLK * 4)], ef_blk)

      def rowloop(rr, c2):
        for g in range(5):
          fb = rr * CH + g * 16
          et16 = et_a[pl.ds(fb, 16)]
          tg16 = tgt_a[pl.ds(fb, 16)]
          key_a[rr, pl.ds(g * 16, 16)] = et16 * n + tg16
          acc = plsc.load_gather(ef_blk, [lane4 + fb * 4])
          for j in range(1, 4):
            acc = acc + plsc.load_gather(ef_blk, [lane4 + (fb * 4 + j)])
          wabs_a[rr, pl.ds(g * 16, 16)] = jnp.abs(acc * 0.25)
        return c2

      lax.fori_loop(0, CPB, rowloop, 0)
      descs = [
          pltpu.async_copy(wabs_a.at[j], deg_sp.at[key_a.at[j]], sem_a,
                           add=True)
          for j in range(CPB)
      ]
      for desc in descs:
        desc.wait()
      return carry

    lax.fori_loop(0, nab, ablock, 0)
    plsc.subcore_barrier()

    base = wid * epw

    def cpipe(j, carry):
      rr = lax.rem(j, CPB)
      b = lax.div(j, CPB)
      par = lax.rem(j, 2)

      @pl.when(j >= 2)
      def _():
        @pl.when(par == 0)
        def _():
          pltpu.make_async_copy(rows0, out_sp.at[srow0], sem_s0).wait()

        @pl.when(par == 1)
        def _():
          pltpu.make_async_copy(rows1, out_sp.at[srow1], sem_s1).wait()

      @pl.when(j < nch)
      def _():
        @pl.when(rr == 0)
        def _():
          bb = base + b * BLK
          pltpu.sync_copy(et_h.at[pl.ds(bb, BLK)], et_a)
          pltpu.sync_copy(tgt_h.at[pl.ds(bb, BLK)], tgt_a)
          pltpu.sync_copy(ef_h.at[pl.ds(bb * 4, BLK * 4)], ef_blk)

        def front(rowsb, c80b, srowb, sem_gb):
          for g in range(5):
            fb = rr * CH + g * 16
            et16 = et_a[pl.ds(fb, 16)]
            tg16 = tgt_a[pl.ds(fb, 16)]
            key_a[rr, pl.ds(g * 16, 16)] = et16 * n + tg16
            acc = plsc.load_gather(ef_blk, [lane4 + fb * 4])
            for jj in range(1, 4):
              acc = acc + plsc.load_gather(ef_blk, [lane4 + (fb * 4 + jj)])
            c80b[pl.ds(g * 16, 16)] = acc * 0.25
          pltpu.sync_copy(deg_sp.at[key_a.at[rr]], deg80)
          for g in range(5):
            sl = pl.ds(g * 16, 16)
            c80b[sl] = c80b[sl] / (deg80[sl] + 1e-8)
          pltpu.async_copy(row_h.at[pl.ds(base + j * CH, CH)], srowb, sem_gb)
          pltpu.async_copy(trel.at[key_a.at[rr]], rowsb, sem_gb)

        @pl.when(par == 0)
        def _():
          front(rows0, c80_0, srow0, sem_g0)

        @pl.when(par == 1)
        def _():
          front(rows1, c80_1, srow1, sem_g1)

      @pl.when(j >= 1)
      def _():
        rp = lax.rem(j - 1, CPB)

        def back(rowsb, c80b, srowb, sem_gb, sem_sb):
          pltpu.make_async_copy(row_h.at[pl.ds(base + (j - 1) * CH, CH)],
                                srowb, sem_gb).wait()
          pltpu.make_async_copy(trel.at[key_a.at[rp]], rowsb, sem_gb).wait()

          def scale(g, c3):
            c16 = c80b[pl.ds(g * 16, 16)]
            for k in range(16):
              spl = jnp.take(c16, jnp.full((16,), k, jnp.int32), mode="fill")
              erow = g * 16 + k
              for h in range(d // 16):
                sl = pl.ds(h * 16, 16)
                rowsb[erow, sl] = rowsb[erow, sl] * spl
            return c3

          lax.fori_loop(0, CH // 16, scale, 0)
          pltpu.async_copy(rowsb, out_sp.at[srowb], sem_sb, add=True)

        @pl.when(par == 1)
        def _():
          back(rows0, c80_0, srow0, sem_g0, sem_s0)

        @pl.when(par == 0)
        def _():
          back(rows1, c80_1, srow1, sem_g1, sem_s1)

      return carry

    lax.fori_loop(0, nch + 1, cpipe, 0)
    if (nch - 1) % 2 == 0:
      pltpu.make_async_copy(rows0, out_sp.at[srow0], sem_s0).wait()
    else:
      pltpu.make_async_copy(rows1, out_sp.at[srow1], sem_s1).wait()
    plsc.subcore_barrier()

    def wb(k, carry):
      ch_id = sid + k * NS

      @pl.when(ch_id < nrc)
      def _():
        pltpu.sync_copy(out_sp.at[pl.ds(ch_id * CH, CH)], rows0)
        pltpu.sync_copy(rows0, out_hbm.at[cid, pl.ds(ch_id * CH, CH)])

      return carry

    lax.fori_loop(0, -(-nrc // NS), wb, 0)

  return sc_kernel


def kernel(node_features, edge_features, adj_e, adj_v, T, edge_index,
           edge_type, W_rel, W_self, b_self):
  n = adj_v.shape[0]
  e = edge_index.shape[1]
  din = node_features.shape[1]
  dout = W_self.shape[0]
  r = W_rel.shape[0]

  w_stack = jnp.concatenate([W_rel, W_self[None]], axis=0)
  bm = 2000
  t_all = pl.pallas_call(
      functools.partial(_transform_body, num_rel=r),
      grid=(r + 1, n // bm),
      in_specs=[
          pl.BlockSpec((bm, din), lambda q, i: (i, 0)),
          pl.BlockSpec((1, dout, din), lambda q, i: (q, 0, 0)),
          pl.BlockSpec((1, dout), lambda q, i: (0, 0)),
      ],
      out_specs=pl.BlockSpec((1, bm, dout), lambda q, i: (q, i, 0)),
      out_shape=jax.ShapeDtypeStruct((r + 1, n, dout), jnp.float32),
  )(node_features, w_stack, b_self.reshape(1, dout))

  trel = t_all[:r].reshape(r * n, dout)
  self_out = t_all[r]

  row_flat = edge_index[0]
  tgt_flat = edge_index[1]
  ef_flat = edge_features.reshape(-1)
  partials = _make_sc_kernel(n, e, dout, r)(
      trel, edge_type, tgt_flat, row_flat, ef_flat)

  out = pl.pallas_call(
      _combine_body,
      grid=(n // bm,),
      in_specs=[
          pl.BlockSpec((NC, bm, dout), lambda i: (0, i, 0)),
          pl.BlockSpec((bm, dout), lambda i: (i, 0)),
      ],
      out_specs=pl.BlockSpec((bm, dout), lambda i: (i, 0)),
      out_shape=jax.ShapeDtypeStruct((n, dout), jnp.float32),
  )(partials, self_out)

  return out, edge_features


# speedup vs baseline: 30.6742x; 1.0450x over previous
import functools

import jax
import jax.numpy as jnp
from jax import lax
from jax.experimental import pallas as pl
from jax.experimental.pallas import tpu as pltpu
from jax.experimental.pallas import tpu_sc as plsc

NC = 2
NS = 16
NW = NC * NS

CH = 80
BLK = 2000
CPB = BLK // CH


def _rel_transform_body(x_ref, w_ref, o_ref):
  o_ref[...] = lax.dot_general(
      x_ref[...], w_ref[0],
      dimension_numbers=(((1,), (1,)), ((), ())),
      preferred_element_type=jnp.float32,
  )


def _self_transform_body(x_ref, w_ref, b_ref, o_ref):
  o_ref[...] = lax.dot_general(
      x_ref[...], w_ref[...],
      dimension_numbers=(((1,), (1,)), ((), ())),
      preferred_element_type=jnp.float32,
  ) + b_ref[...]


def _combine_body(p_ref, s_ref, o_ref):
  o_ref[...] = p_ref[0] + p_ref[1] + s_ref[...]


def _make_sc_kernel(n, e, d, r):
  epw = e // NW
  nch = epw // CH
  nbl = epw // BLK
  ept = e // NS
  nab = ept // BLK
  deg_sz = r * n
  nrc = n // CH

  mesh = plsc.VectorSubcoreMesh(core_axis_name="c", subcore_axis_name="s")

  @functools.partial(
      pl.kernel,
      mesh=mesh,
      compiler_params=pltpu.CompilerParams(needs_layout_passes=False),
      out_type=jax.ShapeDtypeStruct((NC, n, d), jnp.float32),
      scratch_types=[
          pltpu.VMEM_SHARED((deg_sz,), jnp.float32),
          pltpu.VMEM_SHARED((n, d), jnp.float32),
          pltpu.VMEM((BLK * 4,), jnp.float32),
          pltpu.VMEM((BLK,), jnp.int32),
          pltpu.VMEM((BLK,), jnp.int32),
          pltpu.VMEM((CPB, CH), jnp.int32),
          pltpu.VMEM((CPB, CH), jnp.float32),
          pltpu.VMEM((CH, d), jnp.float32),
          pltpu.VMEM((CH, d), jnp.float32),
          pltpu.VMEM((CH,), jnp.float32),
          pltpu.VMEM((CH,), jnp.float32),
          pltpu.VMEM((CH,), jnp.int32),
          pltpu.VMEM((CH,), jnp.int32),
          pltpu.VMEM((CH,), jnp.float32),
          pltpu.SemaphoreType.DMA,
          pltpu.SemaphoreType.DMA,
          pltpu.SemaphoreType.DMA,
          pltpu.SemaphoreType.DMA,
          pltpu.SemaphoreType.DMA,
      ],
  )
  def sc_kernel(trel, et_h, tgt_h, row_h, ef_h, out_hbm,
                deg_sp, out_sp,
                ef_blk, et_a, tgt_a, key_a, wabs_a,
                rows0, rows1, c80_0, c80_1, srow0, srow1, deg80,
                sem_g0, sem_g1, sem_s0, sem_s1, sem_a):
    cid = lax.axis_index("c")
    sid = lax.axis_index("s")
    wid = sid * NC + cid
    lane4 = lax.iota(jnp.int32, 16) * 4

    def zrow(i, carry):
      for h in range(d // 16):
        rows0[i, pl.ds(h * 16, 16)] = jnp.zeros((16,), jnp.float32)
      return carry

    lax.fori_loop(0, CH, zrow, 0)

    def zout(k, carry):
      ch_id = sid + k * NS

      @pl.when(ch_id < nrc)
      def _():
        pltpu.sync_copy(rows0, out_sp.at[pl.ds(ch_id * CH, CH)])

      return carry

    lax.fori_loop(0, -(-nrc // NS), zout, 0)

    def zc(i, carry):
      ef_blk[pl.ds(i * 16, 16)] = jnp.zeros((16,), jnp.float32)
      return carry

    lax.fori_loop(0, 250, zc, 0)

    @pl.when(sid < 10)
    def _():
      pltpu.sync_copy(ef_blk.at[pl.ds(0, 4000)],
                      deg_sp.at[pl.ds(sid * 4000, 4000)])

    plsc.subcore_barrier()

    def ablock(b, carry):
      abase = sid * ept + b * BLK
      pltpu.sync_copy(et_h.at[pl.ds(abase, BLK)], et_a)
      pltpu.sync_copy(tgt_h.at[pl.ds(abase, BLK)], tgt_a)
      pltpu.sync_copy(ef_h.at[pl.ds(abase * 4, BLK * 4)], ef_blk)

      def rowloop(rr, c2):
        for g in range(5):
          fb = rr * CH + g * 16
          et16 = et_a[pl.ds(fb, 16)]
          tg16 = tgt_a[pl.ds(fb, 16)]
          key_a[rr, pl.ds(g * 16, 16)] = et16 * n + tg16
          acc = plsc.load_gather(ef_blk, [lane4 + fb * 4])
          for j in range(1, 4):
            acc = acc + plsc.load_gather(ef_blk, [lane4 + (fb * 4 + j)])
          wabs_a[rr, pl.ds(g * 16, 16)] = jnp.abs(acc * 0.25)
        return c2

      lax.fori_loop(0, CPB, rowloop, 0)
      descs = [
          pltpu.async_copy(wabs_a.at[j], deg_sp.at[key_a.at[j]], sem_a,
                           add=True)
          for j in range(CPB)
      ]
      for desc in descs:
        desc.wait()
      return carry

    lax.fori_loop(0, nab, ablock, 0)
    plsc.subcore_barrier()

    base = wid * epw

    def cpipe(j, carry):
      rr = lax.rem(j, CPB)
      b = lax.div(j, CPB)
      par = lax.rem(j, 2)

      @pl.when(j >= 2)
      def _():
        @pl.when(par == 0)
        def _():
          pltpu.make_async_copy(rows0, out_sp.at[srow0], sem_s0).wait()

        @pl.when(par == 1)
        def _():
          pltpu.make_async_copy(rows1, out_sp.at[srow1], sem_s1).wait()

      @pl.when(j < nch)
      def _():
        @pl.when(rr == 0)
        def _():
          bb = base + b * BLK
          pltpu.sync_copy(et_h.at[pl.ds(bb, BLK)], et_a)
          pltpu.sync_copy(tgt_h.at[pl.ds(bb, BLK)], tgt_a)
          pltpu.sync_copy(ef_h.at[pl.ds(bb * 4, BLK * 4)], ef_blk)

        def front(rowsb, c80b, srowb, sem_gb):
          for g in range(5):
            fb = rr * CH + g * 16
            et16 = et_a[pl.ds(fb, 16)]
            tg16 = tgt_a[pl.ds(fb, 16)]
            key_a[rr, pl.ds(g * 16, 16)] = et16 * n + tg16
            acc = plsc.load_gather(ef_blk, [lane4 + fb * 4])
            for jj in range(1, 4):
              acc = acc + plsc.load_gather(ef_blk, [lane4 + (fb * 4 + jj)])
            c80b[pl.ds(g * 16, 16)] = acc * 0.25
          pltpu.sync_copy(deg_sp.at[key_a.at[rr]], deg80)
          for g in range(5):
            sl = pl.ds(g * 16, 16)
            c80b[sl] = c80b[sl] / (deg80[sl] + 1e-8)
          pltpu.async_copy(row_h.at[pl.ds(base + j * CH, CH)], srowb, sem_gb)
          pltpu.async_copy(trel.at[key_a.at[rr]], rowsb, sem_gb)

        @pl.when(par == 0)
        def _():
          front(rows0, c80_0, srow0, sem_g0)

        @pl.when(par == 1)
        def _():
          front(rows1, c80_1, srow1, sem_g1)

      @pl.when(j >= 1)
      def _():
        rp = lax.rem(j - 1, CPB)

        def back(rowsb, c80b, srowb, sem_gb, sem_sb):
          pltpu.make_async_copy(row_h.at[pl.ds(base + (j - 1) * CH, CH)],
                                srowb, sem_gb).wait()
          pltpu.make_async_copy(trel.at[key_a.at[rp]], rowsb, sem_gb).wait()

          def scale(g, c3):
            c16 = c80b[pl.ds(g * 16, 16)]
            for k in range(16):
              spl = jnp.take(c16, jnp.full((16,), k, jnp.int32), mode="fill")
              erow = g * 16 + k
              for h in range(d // 16):
                sl = pl.ds(h * 16, 16)
                rowsb[erow, sl] = rowsb[erow, sl] * spl
            return c3

          lax.fori_loop(0, CH // 16, scale, 0)
          pltpu.async_copy(rowsb, out_sp.at[srowb], sem_sb, add=True)

        @pl.when(par == 1)
        def _():
          back(rows0, c80_0, srow0, sem_g0, sem_s0)

        @pl.when(par == 0)
        def _():
          back(rows1, c80_1, srow1, sem_g1, sem_s1)

      return carry

    lax.fori_loop(0, nch + 1, cpipe, 0)
    if (nch - 1) % 2 == 0:
      pltpu.make_async_copy(rows0, out_sp.at[srow0], sem_s0).wait()
    else:
      pltpu.make_async_copy(rows1, out_sp.at[srow1], sem_s1).wait()
    plsc.subcore_barrier()

    def wb(k, carry):
      ch_id = sid + k * NS

      @pl.when(ch_id < nrc)
      def _():
        pltpu.sync_copy(out_sp.at[pl.ds(ch_id * CH, CH)], rows0)
        pltpu.sync_copy(rows0, out_hbm.at[cid, pl.ds(ch_id * CH, CH)])

      return carry

    lax.fori_loop(0, -(-nrc // NS), wb, 0)

  return sc_kernel


def kernel(node_features, edge_features, adj_e, adj_v, T, edge_index,
           edge_type, W_rel, W_self, b_self):
  n = adj_v.shape[0]
  e = edge_index.shape[1]
  din = node_features.shape[1]
  dout = W_self.shape[0]
  r = W_rel.shape[0]

  bm = 2000
  nb = n // bm
  trel = pl.pallas_call(
      _rel_transform_body,
      grid=(r, nb),
      in_specs=[
          pl.BlockSpec((bm, din), lambda q, i: (i, 0)),
          pl.BlockSpec((1, dout, din), lambda q, i: (q, 0, 0)),
      ],
      out_specs=pl.BlockSpec((bm, dout), lambda q, i: (q * nb + i, 0)),
      out_shape=jax.ShapeDtypeStruct((r * n, dout), jnp.float32),
  )(node_features, W_rel)

  self_out = pl.pallas_call(
      _self_transform_body,
      grid=(nb,),
      in_specs=[
          pl.BlockSpec((bm, din), lambda i: (i, 0)),
          pl.BlockSpec((dout, din), lambda i: (0, 0)),
          pl.BlockSpec((1, dout), lambda i: (0, 0)),
      ],
      out_specs=pl.BlockSpec((bm, dout), lambda i: (i, 0)),
      out_shape=jax.ShapeDtypeStruct((n, dout), jnp.float32),
  )(node_features, W_self, b_self.reshape(1, dout))

  row_flat = edge_index[0]
  tgt_flat = edge_index[1]
  ef_flat = edge_features.reshape(-1)
  partials = _make_sc_kernel(n, e, dout, r)(
      trel, edge_type, tgt_flat, row_flat, ef_flat)

  out = pl.pallas_call(
      _combine_body,
      grid=(n // bm,),
      in_specs=[
          pl.BlockSpec((NC, bm, dout), lambda i: (0, i, 0)),
          pl.BlockSpec((bm, dout), lambda i: (i, 0)),
      ],
      out_specs=pl.BlockSpec((bm, dout), lambda i: (i, 0)),
      out_shape=jax.ShapeDtypeStruct((n, dout), jnp.float32),
  )(partials, self_out)

  return out, edge_features


# speedup vs baseline: 45.1855x; 1.4731x over previous
import functools

import jax
import jax.numpy as jnp
from jax import lax
from jax.experimental import pallas as pl
from jax.experimental.pallas import tpu as pltpu
from jax.experimental.pallas import tpu_sc as plsc

NC = 2
NS = 16
NW = NC * NS

CH = 128
SBLK = 512
CPB = SBLK // CH
WBC = 80


def _rel_transform_body(x_ref, w_ref, o_ref):
  o_ref[...] = lax.dot_general(
      x_ref[...], w_ref[0],
      dimension_numbers=(((1,), (1,)), ((), ())),
      preferred_element_type=jnp.float32,
  )


def _self_transform_body(x_ref, w_ref, b_ref, o_ref):
  o_ref[...] = lax.dot_general(
      x_ref[...], w_ref[...],
      dimension_numbers=(((1,), (1,)), ((), ())),
      preferred_element_type=jnp.float32,
  ) + b_ref[...]


def _combine_body(p_ref, s_ref, o_ref):
  o_ref[...] = p_ref[0] + p_ref[1] + s_ref[...]


def _make_sc_kernel(n, e, d, r):
  nbt = e // SBLK
  nba = -(-nbt // NS)
  nbw = -(-nbt // NW)
  remw = nbt % NW
  nwmax = nbw * CPB
  deg_sz = r * n
  nrc = n // WBC

  mesh = plsc.VectorSubcoreMesh(core_axis_name="c", subcore_axis_name="s")

  @functools.partial(
      pl.kernel,
      mesh=mesh,
      compiler_params=pltpu.CompilerParams(needs_layout_passes=False),
      out_type=jax.ShapeDtypeStruct((NC, n, d), jnp.float32),
      scratch_types=[
          pltpu.VMEM_SHARED((deg_sz,), jnp.float32),
          pltpu.VMEM_SHARED((n, d), jnp.float32),
          pltpu.VMEM((4, SBLK), jnp.float32),
          pltpu.VMEM((2, SBLK), jnp.int32),
          pltpu.VMEM((SBLK,), jnp.int32),
          pltpu.VMEM((CPB, CH), jnp.int32),
          pltpu.VMEM((CPB, CH), jnp.float32),
          pltpu.VMEM((CH, d), jnp.float32),
          pltpu.VMEM((CH, d), jnp.float32),
          pltpu.VMEM((CH,), jnp.float32),
          pltpu.VMEM((CH,), jnp.float32),
          pltpu.VMEM((CH,), jnp.int32),
          pltpu.VMEM((CH,), jnp.int32),
          pltpu.VMEM((CH,), jnp.float32),
          pltpu.VMEM((2000,), jnp.float32),
          pltpu.SemaphoreType.DMA,
          pltpu.SemaphoreType.DMA,
          pltpu.SemaphoreType.DMA,
          pltpu.SemaphoreType.DMA,
          pltpu.SemaphoreType.DMA,
      ],
  )
  def sc_kernel(trel, et_h, ei_h, ef_t, out_hbm,
                deg_sp, out_sp,
                ef4s, eits, ets, key_a, wabs_a,
                rows0, rows1, c0, c1, srow0, srow1, degc, zbuf,
                sem_g0, sem_g1, sem_s0, sem_s1, sem_a):
    cid = lax.axis_index("c")
    sid = lax.axis_index("s")
    wid = sid * NC + cid
    z16 = jnp.zeros((16,), jnp.float32)

    def zrow(i, carry):
      for h in range(d // 16):
        rows0[i, pl.ds(h * 16, 16)] = z16
      return carry

    lax.fori_loop(0, CH, zrow, 0)

    def zout(k, carry):
      ch_id = sid + k * NS

      @pl.when(ch_id < nrc)
      def _():
        pltpu.sync_copy(rows0.at[pl.ds(0, WBC)],
                        out_sp.at[pl.ds(ch_id * WBC, WBC)])

      return carry

    lax.fori_loop(0, -(-nrc // NS), zout, 0)

    def zc(i, carry):
      zbuf[pl.ds(i * 16, 16)] = z16
      return carry

    lax.fori_loop(0, 125, zc, 0)

    def zdeg(k, carry):
      ch_id = sid + k * NS

      @pl.when(ch_id < deg_sz // 2000)
      def _():
        pltpu.sync_copy(zbuf, deg_sp.at[pl.ds(ch_id * 2000, 2000)])

      return carry

    lax.fori_loop(0, -(-(deg_sz // 2000) // NS), zdeg, 0)
    plsc.subcore_barrier()

    def ablock(k, carry):
      blk = sid + k * NS

      @pl.when(blk < nbt)
      def _():
        off = blk * SBLK
        pltpu.sync_copy(ei_h.at[:, pl.ds(off, SBLK)], eits)
        pltpu.sync_copy(ef_t.at[:, pl.ds(off, SBLK)], ef4s)
        pltpu.sync_copy(et_h.at[pl.ds(off, SBLK)], ets)

        def rowloop(rr, c2):
          for g in range(CH // 16):
            fb = rr * CH + g * 16
            sl = pl.ds(fb, 16)
            key_a[rr, pl.ds(g * 16, 16)] = ets[sl] * n + eits[1, sl]
            w = (ef4s[0, sl] + ef4s[1, sl] + ef4s[2, sl] + ef4s[3, sl])
            wabs_a[rr, pl.ds(g * 16, 16)] = jnp.abs(w * 0.25)
          return c2

        lax.fori_loop(0, CPB, rowloop, 0)
        descs = [
            pltpu.async_copy(wabs_a.at[j], deg_sp.at[key_a.at[j]], sem_a,
                             add=True)
            for j in range(CPB)
        ]
        for desc in descs:
          desc.wait()

      return carry

    lax.fori_loop(0, nba, ablock, 0)
    plsc.subcore_barrier()

    nw = jnp.where(wid < remw, nwmax, nwmax - CPB) if remw else nwmax

    def cpipe(j, carry):
      cc = lax.rem(j, CPB)
      par = lax.rem(j, 2)

      @pl.when(jnp.logical_and(j >= 2, j - 2 < nw))
      def _():
        @pl.when(par == 0)
        def _():
          pltpu.make_async_copy(rows0, out_sp.at[srow0], sem_s0).wait()

        @pl.when(par == 1)
        def _():
          pltpu.make_async_copy(rows1, out_sp.at[srow1], sem_s1).wait()

      @pl.when(j < nw)
      def _():
        @pl.when(cc == 0)
        def _():
          off = (wid + lax.div(j, CPB) * NW) * SBLK
          pltpu.sync_copy(ei_h.at[:, pl.ds(off, SBLK)], eits)
          pltpu.sync_copy(ef_t.at[:, pl.ds(off, SBLK)], ef4s)
          pltpu.sync_copy(et_h.at[pl.ds(off, SBLK)], ets)

        def front(rowsb, cb, srowb, sem_gb):
          for g in range(CH // 16):
            fb = cc * CH + g * 16
            sl = pl.ds(fb, 16)
            gs = pl.ds(g * 16, 16)
            key_a[cc, gs] = ets[sl] * n + eits[1, sl]
            srowb[gs] = eits[0, sl]
            w = (ef4s[0, sl] + ef4s[1, sl] + ef4s[2, sl] + ef4s[3, sl])
            cb[gs] = w * 0.25
          pltpu.sync_copy(deg_sp.at[key_a.at[cc]], degc)
          for g in range(CH // 16):
            gs = pl.ds(g * 16, 16)
            cb[gs] = cb[gs] / (degc[gs] + 1e-8)
          pltpu.async_copy(trel.at[key_a.at[cc]], rowsb, sem_gb)

        @pl.when(par == 0)
        def _():
          front(rows0, c0, srow0, sem_g0)

        @pl.when(par == 1)
        def _():
          front(rows1, c1, srow1, sem_g1)

      @pl.when(jnp.logical_and(j >= 1, j - 1 < nw))
      def _():
        rp = lax.rem(j - 1, CPB)

        def back(rowsb, cb, srowb, sem_gb, sem_sb):
          pltpu.make_async_copy(trel.at[key_a.at[rp]], rowsb, sem_gb).wait()

          def scale(g, c3):
            c16 = cb[pl.ds(g * 16, 16)]
            for k in range(16):
              spl = jnp.take(c16, jnp.full((16,), k, jnp.int32), mode="fill")
              erow = g * 16 + k
              for h in range(d // 16):
                sl = pl.ds(h * 16, 16)
                rowsb[erow, sl] = rowsb[erow, sl] * spl
            return c3

          lax.fori_loop(0, CH // 16, scale, 0)
          pltpu.async_copy(rowsb, out_sp.at[srowb], sem_sb, add=True)

        @pl.when(par == 1)
        def _():
          back(rows0, c0, srow0, sem_g0, sem_s0)

        @pl.when(par == 0)
        def _():
          back(rows1, c1, srow1, sem_g1, sem_s1)

      return carry

    lax.fori_loop(0, nwmax + 2, cpipe, 0)
    plsc.subcore_barrier()

    def wb(k, carry):
      ch_id = sid + k * NS

      @pl.when(ch_id < nrc)
      def _():
        pltpu.sync_copy(out_sp.at[pl.ds(ch_id * WBC, WBC)],
                        rows0.at[pl.ds(0, WBC)])
        pltpu.sync_copy(rows0.at[pl.ds(0, WBC)],
                        out_hbm.at[cid, pl.ds(ch_id * WBC, WBC)])

      return carry

    lax.fori_loop(0, -(-nrc // NS), wb, 0)

  return sc_kernel


def kernel(node_features, edge_features, adj_e, adj_v, T, edge_index,
           edge_type, W_rel, W_self, b_self):
  n = adj_v.shape[0]
  e = edge_index.shape[1]
  din = node_features.shape[1]
  dout = W_self.shape[0]
  r = W_rel.shape[0]

  bm = 2000
  nb = n // bm
  trel = pl.pallas_call(
      _rel_transform_body,
      grid=(r, nb),
      in_specs=[
          pl.BlockSpec((bm, din), lambda q, i: (i, 0)),
          pl.BlockSpec((1, dout, din), lambda q, i: (q, 0, 0)),
      ],
      out_specs=pl.BlockSpec((bm, dout), lambda q, i: (q * nb + i, 0)),
      out_shape=jax.ShapeDtypeStruct((r * n, dout), jnp.float32),
  )(node_features, W_rel)

  self_out = pl.pallas_call(
      _self_transform_body,
      grid=(nb,),
      in_specs=[
          pl.BlockSpec((bm, din), lambda i: (i, 0)),
          pl.BlockSpec((dout, din), lambda i: (0, 0)),
          pl.BlockSpec((1, dout), lambda i: (0, 0)),
      ],
      out_specs=pl.BlockSpec((bm, dout), lambda i: (i, 0)),
      out_shape=jax.ShapeDtypeStruct((n, dout), jnp.float32),
  )(node_features, W_self, b_self.reshape(1, dout))

  ef_t = edge_features.T
  partials = _make_sc_kernel(n, e, dout, r)(
      trel, edge_type, edge_index, ef_t)

  out = pl.pallas_call(
      _combine_body,
      grid=(n // bm,),
      in_specs=[
          pl.BlockSpec((NC, bm, dout), lambda i: (0, i, 0)),
          pl.BlockSpec((bm, dout), lambda i: (i, 0)),
      ],
      out_specs=pl.BlockSpec((bm, dout), lambda i: (i, 0)),
      out_shape=jax.ShapeDtypeStruct((n, dout), jnp.float32),
  )(partials, self_out)

  return out, edge_features
